# baseline (speedup 1.0000x reference)
import functools
import math

import jax
import jax.numpy as jnp
from jax import lax
from jax.experimental import pallas as pl
from jax.experimental.pallas import tpu as pltpu
from jax.experimental.pallas import tpu_sc as plsc

_VOCAB = 100000
_D = 2048
_BATCH = 2
_S = 4096
_NIDS = _BATCH * _S
_SCALE = float(_D) ** 0.5
_NEG_INF = -2.3819763e+38
_HEAD = 256
_NF = _HEAD // 2
_THETA = 10000.0

_NC = 2
_NS = 16
_L = 16
_NW = _NC * _NS
_TC_ROWS = 1024
_SC_ROWS = _NIDS - _TC_ROWS
_BPW = _SC_ROWS // _NW
_CH = 8
_NCH = _BPW // _CH
_NBUF = 4
_VPR = _D // _L


def _make_sc_gather():
  mesh = plsc.VectorSubcoreMesh(core_axis_name="c", subcore_axis_name="s")

  @functools.partial(
      pl.kernel,
      out_type=jax.ShapeDtypeStruct((_SC_ROWS, _D), jnp.float32),
      mesh=mesh,
      scratch_types=[
          pltpu.VMEM((_BPW,), jnp.int32),
      ]
      + [pltpu.VMEM((_CH, _D), jnp.float32)] * _NBUF
      + [pltpu.SemaphoreType.DMA] * (2 * _NBUF),
  )
  def gather_kernel(weight_hbm, ids_hbm, out_hbm, idx_v, *bufs_sems):
    bufs = bufs_sems[:_NBUF]
    gsems = bufs_sems[_NBUF:2 * _NBUF]
    osems = bufs_sems[2 * _NBUF:]
    wid = lax.axis_index("s") * _NC + lax.axis_index("c")
    base = wid * _BPW
    pltpu.sync_copy(ids_hbm.at[pl.ds(base, _BPW)], idx_v)

    def start_gather(ch):
      b = ch % _NBUF
      pltpu.async_copy(
          weight_hbm.at[idx_v.at[pl.ds(ch * _CH, _CH)]], bufs[b], gsems[b])

    def wait_gather(ch):
      b = ch % _NBUF
      pltpu.make_async_copy(
          weight_hbm.at[idx_v.at[pl.ds(ch * _CH, _CH)]], bufs[b],
          gsems[b]).wait()

    def start_out(ch):
      b = ch % _NBUF
      pltpu.async_copy(bufs[b], out_hbm.at[pl.ds(base + ch * _CH, _CH)],
                       osems[b])

    def wait_out(ch):
      b = ch % _NBUF
      pltpu.make_async_copy(bufs[b], out_hbm.at[pl.ds(base + ch * _CH, _CH)],
                            osems[b]).wait()

    def scale(ch):
      buf = bufs[ch % _NBUF]

      @pl.loop(0, _CH)
      def _(r):

        @pl.loop(0, _VPR, unroll=16)
        def _(c):
          buf[r, pl.ds(c * _L, _L)] = buf[r, pl.ds(c * _L, _L)] * _SCALE

    start_gather(0)
    start_gather(1)
    for g in range(_NCH):
      if g + 2 < _NCH:
        if g - 2 >= 0:
          wait_out(g - 2)
        start_gather(g + 2)
      wait_gather(g)
      scale(g)
      start_out(g)
    for ch in range(_NCH - 4, _NCH):
      wait_out(ch)

  return gather_kernel


_sc_gather_cache = []


def _sc_gather(weight, ids):
  if not _sc_gather_cache:
    _sc_gather_cache.append(_make_sc_gather())
  return _sc_gather_cache[0](weight, ids)

_RB = 256


def _mask_body(mask_ref):
  i = pl.program_id(0)
  rows = i * _RB + lax.broadcasted_iota(jnp.int32, (_RB, _S), 0)
  cols = lax.broadcasted_iota(jnp.int32, (_RB, _S), 1)
  mask_ref[0, 0, :, :] = jnp.where(cols > rows, _NEG_INF, 0.0)


_mask = pl.pallas_call(
    _mask_body,
    grid=(_S // _RB,),
    out_specs=pl.BlockSpec((1, 1, _RB, _S), lambda i: (0, 0, i, 0)),
    out_shape=jax.ShapeDtypeStruct((1, 1, _S, _S), jnp.float32),
)


def _tc_gather_body(ids_ref, w_ref, o_ref):
  o_ref[...] = w_ref[...] * _SCALE


_tc_gather = pl.pallas_call(
    _tc_gather_body,
    grid_spec=pltpu.PrefetchScalarGridSpec(
        num_scalar_prefetch=1,
        grid=(_TC_ROWS,),
        in_specs=[pl.BlockSpec((1, 1, _D), lambda i, ids: (ids[i], 0, 0))],
        out_specs=pl.BlockSpec((1, 1, _D), lambda i, ids: (i, 0, 0)),
    ),
    out_shape=jax.ShapeDtypeStruct((_TC_ROWS, 1, _D), jnp.float32),
)


def kernel(input_ids, labels, weight):
  ids = input_ids.reshape(_NIDS)
  sc_part = _sc_gather(weight, ids)
  tc_part = _tc_gather(ids[_SC_ROWS:], weight.reshape(_VOCAB, 1, _D))
  hidden = jnp.concatenate([sc_part, tc_part.reshape(_TC_ROWS, _D)], axis=0)
  mask = _mask()
  inv_freq = 1.0 / (_THETA ** (
      jnp.arange(0, _HEAD, 2, dtype=jnp.float32)[: _NF] / _HEAD))
  t = jnp.arange(_S, dtype=jnp.float32)
  freqs = jnp.exp(1j * jnp.outer(t, inv_freq).astype(jnp.complex64))
  return (hidden.reshape(_BATCH, _S, _D), freqs, mask, labels)


# speedup vs baseline: 8.6812x; 8.6812x over previous
import functools
import math

import jax
import jax.numpy as jnp
from jax import lax
from jax.experimental import pallas as pl
from jax.experimental.pallas import tpu as pltpu
from jax.experimental.pallas import tpu_sc as plsc

_VOCAB = 100000
_D = 2048
_BATCH = 2
_S = 4096
_NIDS = _BATCH * _S
_SCALE = float(_D) ** 0.5
_NEG_INF = -2.3819763e+38
_HEAD = 256
_NF = _HEAD // 2
_THETA = 10000.0

_NC = 2
_NS = 16
_L = 16
_NW = _NC * _NS
_TAIL = 1024
_SC_ROWS = _NIDS - _TAIL
_BPW = _SC_ROWS // _NW
_CH = 8
_NCH = _BPW // _CH
_NBUF = 4
_VPR = _D // _L


def _make_sc_gather():
  mesh = plsc.VectorSubcoreMesh(core_axis_name="c", subcore_axis_name="s")

  @functools.partial(
      pl.kernel,
      out_type=jax.ShapeDtypeStruct((_SC_ROWS, _D), jnp.float32),
      mesh=mesh,
      scratch_types=[
          pltpu.VMEM((_BPW,), jnp.int32),
      ]
      + [pltpu.VMEM((_CH, _D), jnp.float32)] * _NBUF
      + [pltpu.SemaphoreType.DMA] * (2 * _NBUF),
  )
  def gather_kernel(weight_hbm, ids_hbm, out_hbm, idx_v, *bufs_sems):
    bufs = bufs_sems[:_NBUF]
    gsems = bufs_sems[_NBUF:2 * _NBUF]
    osems = bufs_sems[2 * _NBUF:]
    wid = lax.axis_index("s") * _NC + lax.axis_index("c")
    base = wid * _BPW
    pltpu.sync_copy(ids_hbm.at[pl.ds(base, _BPW)], idx_v)

    def start_gather(ch):
      b = ch % _NBUF
      pltpu.async_copy(
          weight_hbm.at[idx_v.at[pl.ds(ch * _CH, _CH)]], bufs[b], gsems[b])

    def wait_gather(ch):
      b = ch % _NBUF
      pltpu.make_async_copy(
          weight_hbm.at[idx_v.at[pl.ds(ch * _CH, _CH)]], bufs[b],
          gsems[b]).wait()

    def start_out(ch):
      b = ch % _NBUF
      pltpu.async_copy(bufs[b], out_hbm.at[pl.ds(base + ch * _CH, _CH)],
                       osems[b])

    def wait_out(ch):
      b = ch % _NBUF
      pltpu.make_async_copy(bufs[b], out_hbm.at[pl.ds(base + ch * _CH, _CH)],
                            osems[b]).wait()

    def scale(ch):
      buf = bufs[ch % _NBUF]

      @pl.loop(0, _CH)
      def _(r):

        @pl.loop(0, _VPR, unroll=16)
        def _(c):
          buf[r, pl.ds(c * _L, _L)] = buf[r, pl.ds(c * _L, _L)] * _SCALE

    start_gather(0)
    start_gather(1)
    for g in range(_NCH):
      if g + 2 < _NCH:
        if g - 2 >= 0:
          wait_out(g - 2)
        start_gather(g + 2)
      wait_gather(g)
      scale(g)
      start_out(g)
    for ch in range(_NCH - 4, _NCH):
      wait_out(ch)

  return gather_kernel


_sc_gather_cache = []


def _sc_gather(weight, ids):
  if not _sc_gather_cache:
    _sc_gather_cache.append(_make_sc_gather())
  return _sc_gather_cache[0](weight, ids)


_RB = 256
_NSTEP = _S // _RB
_RPS = _TAIL // _NSTEP


def _mask_tail_body(ids_ref, w_ref, mask_ref, tail_ref, rbuf, sems):
  i = pl.program_id(0)

  def issue(j):
    b = jax.lax.rem(j, 2)

    @pl.when(j < _NSTEP)
    def _():
      for k in range(_RPS):
        rid = ids_ref[j * _RPS + k]
        pltpu.async_copy(
            w_ref.at[pl.ds(rid, 1), :],
            rbuf.at[b, pl.ds(k, 1), :],
            sems.at[b],
        )

  def drain(j):
    b = jax.lax.rem(j, 2)
    for k in range(_RPS):
      rid = ids_ref[j * _RPS + k]
      pltpu.make_async_copy(
          w_ref.at[pl.ds(rid, 1), :],
          rbuf.at[b, pl.ds(k, 1), :],
          sems.at[b],
      ).wait()

  @pl.when(i == 0)
  def _():
    issue(i)

  issue(i + 1)
  rows = i * _RB + lax.broadcasted_iota(jnp.int32, (_RB, _S), 0)
  cols = lax.broadcasted_iota(jnp.int32, (_RB, _S), 1)
  mask_ref[0, 0, :, :] = jnp.where(cols > rows, _NEG_INF, 0.0)
  drain(i)
  tail_ref[...] = rbuf[lax.rem(i, 2)] * _SCALE


_mask_tail = pl.pallas_call(
    _mask_tail_body,
    grid_spec=pltpu.PrefetchScalarGridSpec(
        num_scalar_prefetch=1,
        grid=(_NSTEP,),
        in_specs=[pl.BlockSpec(memory_space=pltpu.MemorySpace.HBM)],
        out_specs=[
            pl.BlockSpec((1, 1, _RB, _S), lambda i, ids: (0, 0, i, 0)),
            pl.BlockSpec((_RPS, _D), lambda i, ids: (i, 0)),
        ],
        scratch_shapes=[
            pltpu.VMEM((2, _RPS, _D), jnp.float32),
            pltpu.SemaphoreType.DMA((2,)),
        ],
    ),
    out_shape=[
        jax.ShapeDtypeStruct((1, 1, _S, _S), jnp.float32),
        jax.ShapeDtypeStruct((_TAIL, _D), jnp.float32),
    ],
)


def kernel(input_ids, labels, weight):
  ids = input_ids.reshape(_NIDS)
  sc_part = _sc_gather(weight, ids)
  mask, tail = _mask_tail(ids[_SC_ROWS:], weight)
  hidden = jnp.concatenate([sc_part, tail], axis=0)
  inv_freq = 1.0 / (_THETA ** (
      jnp.arange(0, _HEAD, 2, dtype=jnp.float32)[: _NF] / _HEAD))
  t = jnp.arange(_S, dtype=jnp.float32)
  freqs = jnp.exp(1j * jnp.outer(t, inv_freq).astype(jnp.complex64))
  return (hidden.reshape(_BATCH, _S, _D), freqs, mask, labels)


# speedup vs baseline: 13.8587x; 1.5964x over previous
import functools
import math

import jax
import jax.numpy as jnp
from jax import lax
from jax.experimental import pallas as pl
from jax.experimental.pallas import tpu as pltpu
from jax.experimental.pallas import tpu_sc as plsc

_VOCAB = 100000
_D = 2048
_BATCH = 2
_S = 4096
_NIDS = _BATCH * _S
_SCALE = float(_D) ** 0.5
_NEG_INF = -2.3819763e+38
_HEAD = 256
_NF = _HEAD // 2
_THETA = 10000.0

_NC = 2
_NS = 16
_L = 16
_NW = _NC * _NS
_BPW = _NIDS // _NW
_CH = 8
_NCH = _BPW // _CH
_NBUF = 4
_VPR = _D // _L


def _make_sc_gather():
  mesh = plsc.VectorSubcoreMesh(core_axis_name="c", subcore_axis_name="s")

  @functools.partial(
      pl.kernel,
      out_type=jax.ShapeDtypeStruct((_NIDS, _D), jnp.float32),
      mesh=mesh,
      scratch_types=[
          pltpu.VMEM((_BPW,), jnp.int32),
      ]
      + [pltpu.VMEM((_CH, _D), jnp.float32)] * _NBUF
      + [pltpu.SemaphoreType.DMA] * (2 * _NBUF),
  )
  def gather_kernel(weight_hbm, ids_hbm, out_hbm, idx_v, *bufs_sems):
    bufs = bufs_sems[:_NBUF]
    gsems = bufs_sems[_NBUF:2 * _NBUF]
    osems = bufs_sems[2 * _NBUF:]
    wid = lax.axis_index("s") * _NC + lax.axis_index("c")
    base = wid * _BPW
    row = wid // (_S // _BPW)
    col = (wid % (_S // _BPW)) * _BPW
    pltpu.sync_copy(ids_hbm.at[row, pl.ds(col, _BPW)], idx_v)

    def start_gather(ch):
      b = ch % _NBUF
      pltpu.async_copy(
          weight_hbm.at[idx_v.at[pl.ds(ch * _CH, _CH)]], bufs[b], gsems[b])

    def wait_gather(ch):
      b = ch % _NBUF
      pltpu.make_async_copy(
          weight_hbm.at[idx_v.at[pl.ds(ch * _CH, _CH)]], bufs[b],
          gsems[b]).wait()

    def start_out(ch):
      b = ch % _NBUF
      pltpu.async_copy(bufs[b], out_hbm.at[pl.ds(base + ch * _CH, _CH)],
                       osems[b])

    def wait_out(ch):
      b = ch % _NBUF
      pltpu.make_async_copy(bufs[b], out_hbm.at[pl.ds(base + ch * _CH, _CH)],
                            osems[b]).wait()

    def scale(ch):
      buf = bufs[ch % _NBUF]

      @pl.loop(0, _CH)
      def _(r):

        @pl.loop(0, _VPR, unroll=16)
        def _(c):
          buf[r, pl.ds(c * _L, _L)] = buf[r, pl.ds(c * _L, _L)] * _SCALE

    start_gather(0)
    start_gather(1)
    for g in range(_NCH):
      if g + 2 < _NCH:
        if g - 2 >= 0:
          wait_out(g - 2)
        start_gather(g + 2)
      wait_gather(g)
      scale(g)
      start_out(g)
    for ch in range(_NCH - 4, _NCH):
      wait_out(ch)

  return gather_kernel


_sc_gather_cache = []


def _sc_gather(weight, ids):
  if not _sc_gather_cache:
    _sc_gather_cache.append(_make_sc_gather())
  return _sc_gather_cache[0](weight, ids)

_RB = 256


def _mask_body(mask_ref):
  i = pl.program_id(0)
  rows = i * _RB + lax.broadcasted_iota(jnp.int32, (_RB, _S), 0)
  cols = lax.broadcasted_iota(jnp.int32, (_RB, _S), 1)
  mask_ref[0, 0, :, :] = jnp.where(cols > rows, _NEG_INF, 0.0)


_mask = pl.pallas_call(
    _mask_body,
    grid=(_S // _RB,),
    out_specs=pl.BlockSpec((1, 1, _RB, _S), lambda i: (0, 0, i, 0)),
    out_shape=jax.ShapeDtypeStruct((1, 1, _S, _S), jnp.float32),
)


def kernel(input_ids, labels, weight):
  hidden = _sc_gather(weight, input_ids)
  mask = _mask()
  inv_freq = 1.0 / (_THETA ** (
      jnp.arange(0, _HEAD, 2, dtype=jnp.float32)[: _NF] / _HEAD))
  t = jnp.arange(_S, dtype=jnp.float32)
  freqs = jnp.exp(1j * jnp.outer(t, inv_freq).astype(jnp.complex64))
  return (hidden.reshape(_BATCH, _S, _D), freqs, mask, labels)
